# baseline (speedup 1.0000x reference)
import jax
import jax.numpy as jnp
from jax import lax
from jax.experimental import pallas as pl
from jax.experimental.pallas import tpu as pltpu
from jax.experimental.pallas import tpu_sc as plsc

B = 1024
F = 128
A = 100
OUT_W = F + A
ROWS = B * A
PERIOD = 25600
NREP = ROWS // PERIOD
OHT_TILES = 32
OHT_W = A * OHT_TILES
N_PX = PERIOD // B
N_POH = PERIOD // OHT_W
PBc = 3200

NC = 2
NS = 16


def _pretile_body(oht_ref, out_ref):
    for mm in range(OHT_TILES):
        out_ref[:, mm * A:(mm + 1) * A] = oht_ref[...]


def _sc_body(xt_hbm, oht_hbm, pxt_hbm, poht_hbm, xt_sp, oht_sp, sem):
    c = lax.axis_index("c")
    s = lax.axis_index("s")

    rows_per_s = F // NS
    pltpu.sync_copy(xt_hbm.at[pl.ds(s * rows_per_s, rows_per_s)],
                    xt_sp.at[pl.ds(s * rows_per_s, rows_per_s)])
    @pl.when(s < 12)
    def _():
        pltpu.sync_copy(oht_hbm.at[pl.ds(s * 8, 8)],
                        oht_sp.at[pl.ds(s * 8, 8)])
    @pl.when(s == 12)
    def _():
        pltpu.sync_copy(oht_hbm.at[pl.ds(96, 4)],
                        oht_sp.at[pl.ds(96, 4)])
    plsc.subcore_barrier()

    p = s * NC + c
    @pl.when(p < N_PX)
    def _():
        pltpu.async_copy(
            xt_sp, pxt_hbm.at[:, pl.ds(p * B, B)], sem).wait()
    m = (s - 12) * NC + c
    @pl.when((s >= 12) & (m < N_POH))
    def _():
        pltpu.async_copy(
            oht_sp, poht_hbm.at[:, pl.ds(m * OHT_W, OHT_W)], sem).wait()


def _blit_body(pxt_ref, poht_ref, out_ref):
    out_ref[0:F, :] = pxt_ref[...]
    out_ref[F:OUT_W, :] = poht_ref[...]


@jax.jit
def _concat_impl(x, one_hot):
    xt = x.T
    oht = one_hot.T
    ohrep_t = pl.pallas_call(
        _pretile_body,
        out_shape=jax.ShapeDtypeStruct((A, OHT_W), jnp.float32),
    )(oht)

    mesh = plsc.VectorSubcoreMesh(core_axis_name="c", subcore_axis_name="s")
    pxt, poht = pl.kernel(
        _sc_body,
        out_type=(
            jax.ShapeDtypeStruct((F, PERIOD), jnp.float32),
            jax.ShapeDtypeStruct((A, PERIOD), jnp.float32),
        ),
        mesh=mesh,
        scratch_types=[
            pltpu.VMEM_SHARED((F, B), jnp.float32),
            pltpu.VMEM_SHARED((A, OHT_W), jnp.float32),
            pltpu.SemaphoreType.DMA,
        ],
    )(xt, ohrep_t)

    out_t = pl.pallas_call(
        _blit_body,
        grid=(PERIOD // PBc, NREP),
        in_specs=[
            pl.BlockSpec((F, PBc), lambda j, r: (0, j)),
            pl.BlockSpec((A, PBc), lambda j, r: (0, j)),
        ],
        out_specs=pl.BlockSpec(
            (OUT_W, PBc), lambda j, r: (0, r * (PERIOD // PBc) + j)),
        out_shape=jax.ShapeDtypeStruct((OUT_W, ROWS), jnp.float32),
    )(pxt, poht)
    return out_t.T


def kernel(x, one_hot):
    concat_batch = _concat_impl(x, one_hot.astype(x.dtype))
    act = jnp.full((B, A), jnp.nan, dtype=jnp.float32)
    val = jnp.full((B, A), jnp.nan, dtype=jnp.float32)
    mask = jnp.ones((B, A), dtype=bool)
    return (concat_batch, act, val, mask)


# speedup vs baseline: 1.0835x; 1.0835x over previous
import jax
import jax.numpy as jnp
from jax import lax
from jax.experimental import pallas as pl
from jax.experimental.pallas import tpu as pltpu
from jax.experimental.pallas import tpu_sc as plsc

B = 1024
F = 128
A = 100
OUT_W = F + A
ROWS = B * A
PERIOD = 25600
NREP = ROWS // PERIOD
OHT_TILES = 32
OHT_W = A * OHT_TILES
N_PX = PERIOD // B
N_POH = PERIOD // OHT_W
PBc = 6400

NC = 2
NS = 16


def _pretile_body(oht_ref, out_ref):
    for mm in range(OHT_TILES):
        out_ref[:, mm * A:(mm + 1) * A] = oht_ref[...]


def _sc_body(xt_hbm, oht_hbm, pxt_hbm, poht_hbm, xt_sp, oht_sp, sem):
    c = lax.axis_index("c")
    s = lax.axis_index("s")

    rows_per_s = F // NS
    pltpu.sync_copy(xt_hbm.at[pl.ds(s * rows_per_s, rows_per_s)],
                    xt_sp.at[pl.ds(s * rows_per_s, rows_per_s)])
    @pl.when(s < 12)
    def _():
        pltpu.sync_copy(oht_hbm.at[pl.ds(s * 8, 8)],
                        oht_sp.at[pl.ds(s * 8, 8)])
    @pl.when(s == 12)
    def _():
        pltpu.sync_copy(oht_hbm.at[pl.ds(96, 4)],
                        oht_sp.at[pl.ds(96, 4)])
    plsc.subcore_barrier()

    w = s * NC + c
    for k0 in range(2):
        k = w + 32 * k0
        @pl.when(k < 2 * N_PX)
        def _():
            src_col = lax.rem(k, 2) * (B // 2)
            pltpu.async_copy(
                xt_sp.at[:, pl.ds(src_col, B // 2)],
                pxt_hbm.at[:, pl.ds(k * (B // 2), B // 2)], sem).wait()
    m = w - 18
    @pl.when((w >= 18) & (m < N_POH))
    def _():
        pltpu.async_copy(
            oht_sp, poht_hbm.at[:, pl.ds(m * OHT_W, OHT_W)], sem).wait()


def _blit_body(pxt_ref, poht_ref, out_ref):
    out_ref[0:F, :] = pxt_ref[...]
    out_ref[F:OUT_W, :] = poht_ref[...]


@jax.jit
def _concat_impl(x, one_hot):
    xt = x.T
    oht = one_hot.T
    ohrep_t = pl.pallas_call(
        _pretile_body,
        out_shape=jax.ShapeDtypeStruct((A, OHT_W), jnp.float32),
    )(oht)

    mesh = plsc.VectorSubcoreMesh(core_axis_name="c", subcore_axis_name="s")
    pxt, poht = pl.kernel(
        _sc_body,
        out_type=(
            jax.ShapeDtypeStruct((F, PERIOD), jnp.float32),
            jax.ShapeDtypeStruct((A, PERIOD), jnp.float32),
        ),
        mesh=mesh,
        scratch_types=[
            pltpu.VMEM_SHARED((F, B), jnp.float32),
            pltpu.VMEM_SHARED((A, OHT_W), jnp.float32),
            pltpu.SemaphoreType.DMA,
        ],
    )(xt, ohrep_t)

    out_t = pl.pallas_call(
        _blit_body,
        grid=(PERIOD // PBc, NREP),
        in_specs=[
            pl.BlockSpec((F, PBc), lambda j, r: (0, j)),
            pl.BlockSpec((A, PBc), lambda j, r: (0, j)),
        ],
        out_specs=pl.BlockSpec(
            (OUT_W, PBc), lambda j, r: (0, r * (PERIOD // PBc) + j)),
        out_shape=jax.ShapeDtypeStruct((OUT_W, ROWS), jnp.float32),
    )(pxt, poht)
    return out_t.T


def kernel(x, one_hot):
    concat_batch = _concat_impl(x, one_hot.astype(x.dtype))
    act = jnp.full((B, A), jnp.nan, dtype=jnp.float32)
    val = jnp.full((B, A), jnp.nan, dtype=jnp.float32)
    mask = jnp.ones((B, A), dtype=bool)
    return (concat_batch, act, val, mask)


# speedup vs baseline: 1.1289x; 1.0420x over previous
import jax
import jax.numpy as jnp
from jax import lax
from jax.experimental import pallas as pl
from jax.experimental.pallas import tpu as pltpu
from jax.experimental.pallas import tpu_sc as plsc

B = 1024
F = 128
A = 100
OUT_W = F + A
ROWS = B * A
PERIOD = 25600
NREP = ROWS // PERIOD
OHT_TILES = 32
OHT_W = A * OHT_TILES
N_PX = PERIOD // B
N_POH = PERIOD // OHT_W
PBc = 12800

NC = 2
NS = 16


def _pretile_body(oht_ref, out_ref):
    for mm in range(OHT_TILES):
        out_ref[:, mm * A:(mm + 1) * A] = oht_ref[...]


def _sc_body(xt_hbm, oht_hbm, pxt_hbm, poht_hbm, xt_sp, oht_sp, sem):
    c = lax.axis_index("c")
    s = lax.axis_index("s")

    rows_per_s = F // NS
    pltpu.sync_copy(xt_hbm.at[pl.ds(s * rows_per_s, rows_per_s)],
                    xt_sp.at[pl.ds(s * rows_per_s, rows_per_s)])
    @pl.when(s < 12)
    def _():
        pltpu.sync_copy(oht_hbm.at[pl.ds(s * 8, 8)],
                        oht_sp.at[pl.ds(s * 8, 8)])
    @pl.when(s == 12)
    def _():
        pltpu.sync_copy(oht_hbm.at[pl.ds(96, 4)],
                        oht_sp.at[pl.ds(96, 4)])
    plsc.subcore_barrier()

    w = s * NC + c
    for k0 in range(2):
        k = w + 32 * k0
        @pl.when(k < 2 * N_PX)
        def _():
            src_col = lax.rem(k, 2) * (B // 2)
            pltpu.async_copy(
                xt_sp.at[:, pl.ds(src_col, B // 2)],
                pxt_hbm.at[:, pl.ds(k * (B // 2), B // 2)], sem)
    m = w - 18
    @pl.when((w >= 18) & (m < N_POH))
    def _():
        pltpu.async_copy(
            oht_sp, poht_hbm.at[:, pl.ds(m * OHT_W, OHT_W)], sem)
    for k0 in range(2):
        k = w + 32 * k0
        @pl.when(k < 2 * N_PX)
        def _():
            pltpu.make_async_copy(
                xt_sp.at[:, pl.ds(0, B // 2)],
                pxt_hbm.at[:, pl.ds(0, B // 2)], sem).wait()
    @pl.when((w >= 18) & (m < N_POH))
    def _():
        pltpu.make_async_copy(
            oht_sp, poht_hbm.at[:, pl.ds(0, OHT_W)], sem).wait()


def _blit_body(pxt_ref, poht_ref, out_ref):
    out_ref[0:F, :] = pxt_ref[...]
    out_ref[F:OUT_W, :] = poht_ref[...]


@jax.jit
def _concat_impl(x, one_hot):
    xt = x.T
    oht = one_hot.T
    ohrep_t = pl.pallas_call(
        _pretile_body,
        out_shape=jax.ShapeDtypeStruct((A, OHT_W), jnp.float32),
    )(oht)

    mesh = plsc.VectorSubcoreMesh(core_axis_name="c", subcore_axis_name="s")
    pxt, poht = pl.kernel(
        _sc_body,
        out_type=(
            jax.ShapeDtypeStruct((F, PERIOD), jnp.float32),
            jax.ShapeDtypeStruct((A, PERIOD), jnp.float32),
        ),
        mesh=mesh,
        scratch_types=[
            pltpu.VMEM_SHARED((F, B), jnp.float32),
            pltpu.VMEM_SHARED((A, OHT_W), jnp.float32),
            pltpu.SemaphoreType.DMA,
        ],
    )(xt, ohrep_t)

    out_t = pl.pallas_call(
        _blit_body,
        grid=(PERIOD // PBc, NREP),
        in_specs=[
            pl.BlockSpec((F, PBc), lambda j, r: (0, j)),
            pl.BlockSpec((A, PBc), lambda j, r: (0, j)),
        ],
        out_specs=pl.BlockSpec(
            (OUT_W, PBc), lambda j, r: (0, r * (PERIOD // PBc) + j)),
        out_shape=jax.ShapeDtypeStruct((OUT_W, ROWS), jnp.float32),
    )(pxt, poht)
    return out_t.T


def kernel(x, one_hot):
    concat_batch = _concat_impl(x, one_hot.astype(x.dtype))
    act = jnp.full((B, A), jnp.nan, dtype=jnp.float32)
    val = jnp.full((B, A), jnp.nan, dtype=jnp.float32)
    mask = jnp.ones((B, A), dtype=bool)
    return (concat_batch, act, val, mask)


# speedup vs baseline: 1.1730x; 1.0390x over previous
import jax
import jax.numpy as jnp
from jax import lax
from jax.experimental import pallas as pl
from jax.experimental.pallas import tpu as pltpu
from jax.experimental.pallas import tpu_sc as plsc

B = 1024
F = 128
A = 100
OUT_W = F + A
ROWS = B * A
PERIOD = 25600
NREP = ROWS // PERIOD
OHT_TILES = 32
OHT_W = A * OHT_TILES
N_PX = PERIOD // B
N_POH = PERIOD // OHT_W
PBc = 12800

NC = 2
NS = 16


def _pretile_body(x_ref, oh_ref, xt_ref, oht_rep_ref):
    xt_ref[...] = x_ref[...].T
    oht = oh_ref[...].T
    for mm in range(OHT_TILES):
        oht_rep_ref[:, mm * A:(mm + 1) * A] = oht


def _sc_body(xt_hbm, oht_hbm, pxt_hbm, poht_hbm, xt_sp, oht_sp, sem):
    c = lax.axis_index("c")
    s = lax.axis_index("s")

    rows_per_s = F // NS
    pltpu.sync_copy(xt_hbm.at[pl.ds(s * rows_per_s, rows_per_s)],
                    xt_sp.at[pl.ds(s * rows_per_s, rows_per_s)])
    @pl.when(s < 12)
    def _():
        pltpu.sync_copy(oht_hbm.at[pl.ds(s * 8, 8)],
                        oht_sp.at[pl.ds(s * 8, 8)])
    @pl.when(s == 12)
    def _():
        pltpu.sync_copy(oht_hbm.at[pl.ds(96, 4)],
                        oht_sp.at[pl.ds(96, 4)])
    plsc.subcore_barrier()

    w = s * NC + c
    for k0 in range(2):
        k = w + 32 * k0
        @pl.when(k < 2 * N_PX)
        def _():
            src_col = lax.rem(k, 2) * (B // 2)
            pltpu.async_copy(
                xt_sp.at[:, pl.ds(src_col, B // 2)],
                pxt_hbm.at[:, pl.ds(k * (B // 2), B // 2)], sem)
    m = w - 18
    @pl.when((w >= 18) & (m < N_POH))
    def _():
        pltpu.async_copy(
            oht_sp, poht_hbm.at[:, pl.ds(m * OHT_W, OHT_W)], sem)
    for k0 in range(2):
        k = w + 32 * k0
        @pl.when(k < 2 * N_PX)
        def _():
            pltpu.make_async_copy(
                xt_sp.at[:, pl.ds(0, B // 2)],
                pxt_hbm.at[:, pl.ds(0, B // 2)], sem).wait()
    @pl.when((w >= 18) & (m < N_POH))
    def _():
        pltpu.make_async_copy(
            oht_sp, poht_hbm.at[:, pl.ds(0, OHT_W)], sem).wait()


def _blit_body(pxt_ref, poht_ref, out_ref):
    out_ref[0:F, :] = pxt_ref[...]
    out_ref[F:OUT_W, :] = poht_ref[...]


@jax.jit
def _concat_impl(x, one_hot):
    xt, ohrep_t = pl.pallas_call(
        _pretile_body,
        out_shape=(
            jax.ShapeDtypeStruct((F, B), jnp.float32),
            jax.ShapeDtypeStruct((A, OHT_W), jnp.float32),
        ),
    )(x, one_hot)

    mesh = plsc.VectorSubcoreMesh(core_axis_name="c", subcore_axis_name="s")
    pxt, poht = pl.kernel(
        _sc_body,
        out_type=(
            jax.ShapeDtypeStruct((F, PERIOD), jnp.float32),
            jax.ShapeDtypeStruct((A, PERIOD), jnp.float32),
        ),
        mesh=mesh,
        scratch_types=[
            pltpu.VMEM_SHARED((F, B), jnp.float32),
            pltpu.VMEM_SHARED((A, OHT_W), jnp.float32),
            pltpu.SemaphoreType.DMA,
        ],
    )(xt, ohrep_t)

    out_t = pl.pallas_call(
        _blit_body,
        grid=(PERIOD // PBc, NREP),
        in_specs=[
            pl.BlockSpec((F, PBc), lambda j, r: (0, j)),
            pl.BlockSpec((A, PBc), lambda j, r: (0, j)),
        ],
        out_specs=pl.BlockSpec(
            (OUT_W, PBc), lambda j, r: (0, r * (PERIOD // PBc) + j)),
        out_shape=jax.ShapeDtypeStruct((OUT_W, ROWS), jnp.float32),
    )(pxt, poht)
    return out_t.T


def kernel(x, one_hot):
    concat_batch = _concat_impl(x, one_hot.astype(x.dtype))
    act = jnp.full((B, A), jnp.nan, dtype=jnp.float32)
    val = jnp.full((B, A), jnp.nan, dtype=jnp.float32)
    mask = jnp.ones((B, A), dtype=bool)
    return (concat_batch, act, val, mask)
